# Initial kernel scaffold; baseline (speedup 1.0000x reference)
import functools

import jax
import jax.numpy as jnp
from jax import lax
from jax.experimental import pallas as pl
from jax.experimental.pallas import tpu as pltpu
from jax.experimental.pallas import tpu_sc as plsc

B = 16384
S = 200
D = 64
H = 128
NCLS = 2

NC = 2
NS = 16
NW = NC * NS
RPW = B // NW
BLK = 32
NBLK = RPW // BLK
HALF1 = 104
HALF2 = S - HALF1


def _sc_pool_sums(x_flat, emb):
    mesh = plsc.VectorSubcoreMesh(core_axis_name="c", subcore_axis_name="s")

    @functools.partial(
        pl.kernel,
        out_type=jax.ShapeDtypeStruct((B, D), jnp.float32),
        mesh=mesh,
        scratch_types=[
            pltpu.VMEM((2, BLK * S), jnp.int32),
            pltpu.VMEM((2, S, D), jnp.float32),
            pltpu.VMEM((2, BLK, D), jnp.float32),
            pltpu.SemaphoreType.DMA,
            pltpu.SemaphoreType.DMA,
            pltpu.SemaphoreType.DMA,
            pltpu.SemaphoreType.DMA,
            pltpu.SemaphoreType.DMA,
            pltpu.SemaphoreType.DMA,
        ],
    )
    def pool(x_hbm, emb_hbm, out_hbm, idx_v, rows_v, out_v,
             sem_i0, sem_i1, sem_g0, sem_g1, sem_o0, sem_o1):
        wid = lax.axis_index("s") * NC + lax.axis_index("c")
        base = wid * RPW
        sems_i = (sem_i0, sem_i1)
        sems_g = (sem_g0, sem_g1)
        sems_o = (sem_o0, sem_o1)

        def idx_copy(bsl, blk):
            return pltpu.make_async_copy(
                x_hbm.at[pl.ds((base + blk * BLK) * S, BLK * S)],
                idx_v.at[bsl],
                sems_i[bsl])

        def gather_pair(bsl, r, rsl):
            off = r * S
            c1 = pltpu.make_async_copy(
                emb_hbm.at[idx_v.at[bsl, pl.ds(off, HALF1)]],
                rows_v.at[rsl, pl.ds(0, HALF1)],
                sems_g[rsl])
            c2 = pltpu.make_async_copy(
                emb_hbm.at[idx_v.at[bsl, pl.ds(off + HALF1, HALF2)]],
                rows_v.at[rsl, pl.ds(HALF1, HALF2)],
                sems_g[rsl])
            return c1, c2

        def out_copy(osl, blk):
            return pltpu.make_async_copy(
                out_v.at[osl],
                out_hbm.at[pl.ds(base + blk * BLK, BLK)],
                sems_o[osl])

        def accum_row(rsl, osl, rloc):
            def bodyf(i, carry):
                accs = list(carry)
                for j in range(8):
                    rr = i * 8 + j
                    for c in range(4):
                        k = c * 2 + (j % 2)
                        accs[k] = accs[k] + rows_v[rsl, rr, pl.ds(c * 16, 16)]
                return tuple(accs)
            z = jnp.zeros((16,), jnp.float32)
            accs = lax.fori_loop(0, S // 8, bodyf, (z,) * 8)
            for c in range(4):
                out_v[osl, rloc, pl.ds(c * 16, 16)] = accs[2 * c] + accs[2 * c + 1]

        def block_body(blk, bsl, first, last):
            if not first:
                @pl.when(blk >= 2)
                def _():
                    out_copy(bsl, blk - 2).wait()

            def pair(p, _):
                for sl in (0, 1):
                    r = 2 * p + sl

                    @pl.when(r + 1 < BLK)
                    def _():
                        n1, n2 = gather_pair(bsl, r + 1, 1 - sl)
                        n1.start()
                        n2.start()

                    c1, c2 = gather_pair(bsl, r, sl)
                    c1.wait()
                    c2.wait()
                    accum_row(sl, bsl, r)
                return _
            lax.fori_loop(0, BLK // 2, pair, None)

            if not last:
                @pl.when(blk + 2 < NBLK)
                def _():
                    idx_copy(bsl, blk + 2).start()

                idx_copy(1 - bsl, blk + 1).wait()
                g1, g2 = gather_pair(1 - bsl, 0, 0)
                g1.start()
                g2.start()

            out_copy(bsl, blk).start()

        c = idx_copy(0, 0)
        c.start()
        c.wait()
        idx_copy(1, 1).start()
        g1, g2 = gather_pair(0, 0, 0)
        g1.start()
        g2.start()

        def superblock(i, _):
            blk0 = 2 * i
            block_body(blk0, 0, first=False, last=False)
            block_body(blk0 + 1, 1, first=False, last=False)
            return _

        block_body(0, 0, first=True, last=False)
        block_body(1, 1, first=False, last=False)
        lax.fori_loop(1, NBLK // 2 - 1, superblock, None)
        block_body(NBLK - 2, 0, first=False, last=False)
        block_body(NBLK - 1, 1, first=False, last=True)

        out_copy(0, NBLK - 2).wait()
        out_copy(1, NBLK - 1).wait()

    return pool(x_flat, emb)


def _tc_head(x, sums, W1, b1, W2, b2):
    BB = 2048

    def mlp_body(x_ref, s_ref, w1_ref, b1_ref, w2_ref, b2_ref, o_ref):
        xb = x_ref[...]
        lens = jnp.sum((xb != 0).astype(jnp.float32), axis=1, keepdims=True)
        pooled = s_ref[...] / jnp.maximum(lens, 1.0)
        h = jnp.dot(pooled, w1_ref[...], preferred_element_type=jnp.float32)
        h = jnp.maximum(h + b1_ref[...], 0.0)
        o_ref[...] = (jnp.dot(h, w2_ref[...], preferred_element_type=jnp.float32)
                      + b2_ref[...])

    return pl.pallas_call(
        mlp_body,
        grid=(B // BB,),
        in_specs=[
            pl.BlockSpec((BB, S), lambda i: (i, 0)),
            pl.BlockSpec((BB, D), lambda i: (i, 0)),
            pl.BlockSpec((D, H), lambda i: (0, 0)),
            pl.BlockSpec((1, H), lambda i: (0, 0)),
            pl.BlockSpec((H, NCLS), lambda i: (0, 0)),
            pl.BlockSpec((1, NCLS), lambda i: (0, 0)),
        ],
        out_specs=pl.BlockSpec((BB, NCLS), lambda i: (i, 0)),
        out_shape=jax.ShapeDtypeStruct((B, NCLS), jnp.float32),
    )(x, sums, W1, b1.reshape(1, H), W2, b2.reshape(1, NCLS))


def kernel(x, emb, W1, b1, W2, b2):
    x = x.astype(jnp.int32)
    sums = _sc_pool_sums(x.reshape(-1), emb)
    return _tc_head(x, sums, W1, b1, W2, b2)


# speedup vs baseline: 3.2415x; 3.2415x over previous
import functools

import jax
import jax.numpy as jnp
from jax import lax
from jax.experimental import pallas as pl
from jax.experimental.pallas import tpu as pltpu
from jax.experimental.pallas import tpu_sc as plsc

B = 16384
S = 200
D = 64
H = 128
NCLS = 2

NC = 2
NS = 16
NW = NC * NS
RPW = B // NW
BLK = 32
NBLK = RPW // BLK
HALF1 = 104
HALF2 = S - HALF1


def _sc_pool_sums(x_flat, emb):
    mesh = plsc.VectorSubcoreMesh(core_axis_name="c", subcore_axis_name="s",
                                  num_cores=NC, num_subcores=NS)

    @functools.partial(
        pl.kernel,
        out_type=jax.ShapeDtypeStruct((B * D,), jnp.float32),
        compiler_params=pltpu.CompilerParams(use_tc_tiling_on_sc=False),
        mesh=mesh,
        scratch_types=[
            pltpu.VMEM((BLK * S,), jnp.int32),
            pltpu.VMEM((BLK * S,), jnp.int32),
            pltpu.VMEM((S, D), jnp.float32),
            pltpu.VMEM((S, D), jnp.float32),
            pltpu.VMEM((BLK * D,), jnp.float32),
            pltpu.VMEM((BLK * D,), jnp.float32),
            pltpu.SemaphoreType.DMA,
            pltpu.SemaphoreType.DMA,
            pltpu.SemaphoreType.DMA,
            pltpu.SemaphoreType.DMA,
            pltpu.SemaphoreType.DMA,
            pltpu.SemaphoreType.DMA,
        ],
    )
    def pool(x_hbm, emb_hbm, out_hbm, idx_v0, idx_v1, rows_v0, rows_v1,
             out_v0, out_v1, sem_i0, sem_i1, sem_g0, sem_g1, sem_o0, sem_o1):
        wid = lax.axis_index("s") * NC + lax.axis_index("c")
        base = wid * RPW
        idxs = (idx_v0, idx_v1)
        rows = (rows_v0, rows_v1)
        outs = (out_v0, out_v1)
        sems_i = (sem_i0, sem_i1)
        sems_g = (sem_g0, sem_g1)
        sems_o = (sem_o0, sem_o1)

        def idx_copy(bsl, blk):
            return pltpu.make_async_copy(
                x_hbm.at[pl.ds((base + blk * BLK) * S, BLK * S)],
                idxs[bsl],
                sems_i[bsl])

        def gather_pair(bsl, r, rsl):
            off = r * S
            c1 = pltpu.make_async_copy(
                emb_hbm.at[idxs[bsl].at[pl.ds(off, HALF1)]],
                rows[rsl].at[pl.ds(0, HALF1)],
                sems_g[rsl])
            c2 = pltpu.make_async_copy(
                emb_hbm.at[idxs[bsl].at[pl.ds(off + HALF1, HALF2)]],
                rows[rsl].at[pl.ds(HALF1, HALF2)],
                sems_g[rsl])
            return c1, c2

        def out_copy(osl, blk):
            return pltpu.make_async_copy(
                outs[osl],
                out_hbm.at[pl.ds((base + blk * BLK) * D, BLK * D)],
                sems_o[osl])

        def accum_row(rsl, osl, rloc):
            rv = rows[rsl]
            def bodyf(i, carry):
                accs = list(carry)
                for j in range(8):
                    rr = i * 8 + j
                    for c in range(4):
                        k = c * 2 + (j % 2)
                        accs[k] = accs[k] + rv[rr, pl.ds(c * 16, 16)]
                return tuple(accs)
            z = jnp.zeros((16,), jnp.float32)
            accs = lax.fori_loop(0, S // 8, bodyf, (z,) * 8)
            ov = outs[osl]
            for c in range(4):
                ov[pl.ds(rloc * D + c * 16, 16)] = accs[2 * c] + accs[2 * c + 1]

        def block_body(blk, bsl, first, last):
            if not first:
                @pl.when(blk >= 2)
                def _():
                    out_copy(bsl, blk - 2).wait()

            def pair(p, _):
                for sl in (0, 1):
                    r = 2 * p + sl

                    @pl.when(r + 1 < BLK)
                    def _():
                        n1, n2 = gather_pair(bsl, r + 1, 1 - sl)
                        n1.start()
                        n2.start()

                    c1, c2 = gather_pair(bsl, r, sl)
                    c1.wait()
                    c2.wait()
                    accum_row(sl, bsl, r)
                return _
            lax.fori_loop(0, BLK // 2, pair, None)

            if not last:
                @pl.when(blk + 2 < NBLK)
                def _():
                    idx_copy(bsl, blk + 2).start()

                idx_copy(1 - bsl, blk + 1).wait()
                g1, g2 = gather_pair(1 - bsl, 0, 0)
                g1.start()
                g2.start()

            out_copy(bsl, blk).start()

        c = idx_copy(0, 0)
        c.start()
        c.wait()
        idx_copy(1, 1).start()
        g1, g2 = gather_pair(0, 0, 0)
        g1.start()
        g2.start()

        def superblock(i, _):
            blk0 = 2 * i
            block_body(blk0, 0, first=False, last=False)
            block_body(blk0 + 1, 1, first=False, last=False)
            return _

        block_body(0, 0, first=True, last=False)
        block_body(1, 1, first=False, last=False)
        lax.fori_loop(1, NBLK // 2 - 1, superblock, None)
        block_body(NBLK - 2, 0, first=False, last=False)
        block_body(NBLK - 1, 1, first=False, last=True)

        out_copy(0, NBLK - 2).wait()
        out_copy(1, NBLK - 1).wait()

    return pool(x_flat, emb)


def _tc_head(x, sums, W1, b1, W2, b2):
    BB = 2048

    def mlp_body(x_ref, s_ref, w1_ref, b1_ref, w2_ref, b2_ref, o_ref):
        xb = x_ref[...]
        lens = jnp.sum((xb != 0).astype(jnp.float32), axis=1, keepdims=True)
        pooled = s_ref[...] / jnp.maximum(lens, 1.0)
        h = jnp.dot(pooled, w1_ref[...], preferred_element_type=jnp.float32)
        h = jnp.maximum(h + b1_ref[...], 0.0)
        o_ref[...] = (jnp.dot(h, w2_ref[...], preferred_element_type=jnp.float32)
                      + b2_ref[...])

    return pl.pallas_call(
        mlp_body,
        grid=(B // BB,),
        in_specs=[
            pl.BlockSpec((BB, S), lambda i: (i, 0)),
            pl.BlockSpec((BB, D), lambda i: (i, 0)),
            pl.BlockSpec((D, H), lambda i: (0, 0)),
            pl.BlockSpec((1, H), lambda i: (0, 0)),
            pl.BlockSpec((H, NCLS), lambda i: (0, 0)),
            pl.BlockSpec((1, NCLS), lambda i: (0, 0)),
        ],
        out_specs=pl.BlockSpec((BB, NCLS), lambda i: (i, 0)),
        out_shape=jax.ShapeDtypeStruct((B, NCLS), jnp.float32),
    )(x, sums, W1, b1.reshape(1, H), W2, b2.reshape(1, NCLS))


def kernel(x, emb, W1, b1, W2, b2):
    x = x.astype(jnp.int32)
    sums = _sc_pool_sums(x.reshape(-1), emb).reshape(B, D)
    return _tc_head(x, sums, W1, b1, W2, b2)


# speedup vs baseline: 3.7507x; 1.1571x over previous
import functools

import jax
import jax.numpy as jnp
from jax import lax
from jax.experimental import pallas as pl
from jax.experimental.pallas import tpu as pltpu
from jax.experimental.pallas import tpu_sc as plsc

B = 16384
S = 200
D = 64
H = 128
NCLS = 2

NC = 2
NS = 16
NW = NC * NS
RPW = B // NW
BLK = 32
NBLK = RPW // BLK
HALF1 = 104
HALF2 = S - HALF1


def _sc_pool_sums(x_flat, emb):
    mesh = plsc.VectorSubcoreMesh(core_axis_name="c", subcore_axis_name="s",
                                  num_cores=NC, num_subcores=NS)

    @functools.partial(
        pl.kernel,
        out_type=jax.ShapeDtypeStruct((B * D,), jnp.float32),
        compiler_params=pltpu.CompilerParams(use_tc_tiling_on_sc=False),
        mesh=mesh,
        scratch_types=[
            pltpu.VMEM((BLK * S,), jnp.int32),
            pltpu.VMEM((BLK * S,), jnp.int32),
            pltpu.VMEM((S, D), jnp.float32),
            pltpu.VMEM((S, D), jnp.float32),
            pltpu.VMEM((S, D), jnp.float32),
            pltpu.VMEM((S, D), jnp.float32),
            pltpu.VMEM((BLK * D,), jnp.float32),
            pltpu.VMEM((BLK * D,), jnp.float32),
            pltpu.SemaphoreType.DMA,
            pltpu.SemaphoreType.DMA,
            pltpu.SemaphoreType.DMA,
            pltpu.SemaphoreType.DMA,
            pltpu.SemaphoreType.DMA,
            pltpu.SemaphoreType.DMA,
            pltpu.SemaphoreType.DMA,
            pltpu.SemaphoreType.DMA,
        ],
    )
    def pool(x_hbm, emb_hbm, out_hbm, idx_v0, idx_v1, rows_v0, rows_v1,
             rows_v2, rows_v3, out_v0, out_v1, sem_i0, sem_i1,
             sem_g0, sem_g1, sem_g2, sem_g3, sem_o0, sem_o1):
        wid = lax.axis_index("s") * NC + lax.axis_index("c")
        base = wid * RPW
        idxs = (idx_v0, idx_v1)
        rows = (rows_v0, rows_v1, rows_v2, rows_v3)
        outs = (out_v0, out_v1)
        sems_i = (sem_i0, sem_i1)
        sems_g = (sem_g0, sem_g1, sem_g2, sem_g3)
        sems_o = (sem_o0, sem_o1)

        def idx_copy(bsl, blk):
            return pltpu.make_async_copy(
                x_hbm.at[pl.ds((base + blk * BLK) * S, BLK * S)],
                idxs[bsl],
                sems_i[bsl])

        def gather_pair(bsl, r, rsl):
            off = r * S
            c1 = pltpu.make_async_copy(
                emb_hbm.at[idxs[bsl].at[pl.ds(off, HALF1)]],
                rows[rsl].at[pl.ds(0, HALF1)],
                sems_g[rsl])
            c2 = pltpu.make_async_copy(
                emb_hbm.at[idxs[bsl].at[pl.ds(off + HALF1, HALF2)]],
                rows[rsl].at[pl.ds(HALF1, HALF2)],
                sems_g[rsl])
            return c1, c2

        def out_copy(osl, blk):
            return pltpu.make_async_copy(
                outs[osl],
                out_hbm.at[pl.ds((base + blk * BLK) * D, BLK * D)],
                sems_o[osl])

        def accum_row(rsl, osl, rloc):
            rv = rows[rsl]
            def bodyf(i, carry):
                accs = list(carry)
                for j in range(8):
                    rr = i * 8 + j
                    for c in range(4):
                        k = c * 2 + (j % 2)
                        accs[k] = accs[k] + rv[rr, pl.ds(c * 16, 16)]
                return tuple(accs)
            z = jnp.zeros((16,), jnp.float32)
            accs = lax.fori_loop(0, S // 8, bodyf, (z,) * 8)
            ov = outs[osl]
            for c in range(4):
                ov[pl.ds(rloc * D + c * 16, 16)] = accs[2 * c] + accs[2 * c + 1]

        def block_body(blk, bsl, first, last):
            if not first:
                @pl.when(blk >= 2)
                def _():
                    out_copy(bsl, blk - 2).wait()

            def quad(qi, _):
                for q in range(4):
                    r = 4 * qi + q

                    @pl.when(r + 3 < BLK)
                    def _():
                        n1, n2 = gather_pair(bsl, r + 3, (q + 3) % 4)
                        n1.start()
                        n2.start()

                    c1, c2 = gather_pair(bsl, r, q)
                    c1.wait()
                    c2.wait()
                    accum_row(q, bsl, r)
                return _
            lax.fori_loop(0, BLK // 4, quad, None)

            if not last:
                @pl.when(blk + 2 < NBLK)
                def _():
                    idx_copy(bsl, blk + 2).start()

                idx_copy(1 - bsl, blk + 1).wait()
                for j in range(3):
                    g1, g2 = gather_pair(1 - bsl, j, j)
                    g1.start()
                    g2.start()

            out_copy(bsl, blk).start()

        c = idx_copy(0, 0)
        c.start()
        c.wait()
        idx_copy(1, 1).start()
        for j in range(3):
            g1, g2 = gather_pair(0, j, j)
            g1.start()
            g2.start()

        def superblock(i, _):
            blk0 = 2 * i
            block_body(blk0, 0, first=False, last=False)
            block_body(blk0 + 1, 1, first=False, last=False)
            return _

        block_body(0, 0, first=True, last=False)
        block_body(1, 1, first=False, last=False)
        lax.fori_loop(1, NBLK // 2 - 1, superblock, None)
        block_body(NBLK - 2, 0, first=False, last=False)
        block_body(NBLK - 1, 1, first=False, last=True)

        out_copy(0, NBLK - 2).wait()
        out_copy(1, NBLK - 1).wait()

    return pool(x_flat, emb)


def _tc_head(x, sums, W1, b1, W2, b2):
    BB = 2048

    def mlp_body(x_ref, s_ref, w1_ref, b1_ref, w2_ref, b2_ref, o_ref):
        xb = x_ref[...]
        lens = jnp.sum((xb != 0).astype(jnp.float32), axis=1, keepdims=True)
        pooled = s_ref[...] / jnp.maximum(lens, 1.0)
        h = jnp.dot(pooled, w1_ref[...], preferred_element_type=jnp.float32)
        h = jnp.maximum(h + b1_ref[...], 0.0)
        o_ref[...] = (jnp.dot(h, w2_ref[...], preferred_element_type=jnp.float32)
                      + b2_ref[...])

    return pl.pallas_call(
        mlp_body,
        grid=(B // BB,),
        in_specs=[
            pl.BlockSpec((BB, S), lambda i: (i, 0)),
            pl.BlockSpec((BB, D), lambda i: (i, 0)),
            pl.BlockSpec((D, H), lambda i: (0, 0)),
            pl.BlockSpec((1, H), lambda i: (0, 0)),
            pl.BlockSpec((H, NCLS), lambda i: (0, 0)),
            pl.BlockSpec((1, NCLS), lambda i: (0, 0)),
        ],
        out_specs=pl.BlockSpec((BB, NCLS), lambda i: (i, 0)),
        out_shape=jax.ShapeDtypeStruct((B, NCLS), jnp.float32),
    )(x, sums, W1, b1.reshape(1, H), W2, b2.reshape(1, NCLS))


def kernel(x, emb, W1, b1, W2, b2):
    x = x.astype(jnp.int32)
    sums = _sc_pool_sums(x.reshape(-1), emb).reshape(B, D)
    return _tc_head(x, sums, W1, b1, W2, b2)
